# baseline (speedup 1.0000x reference)
import numpy as np
import jax
import jax.numpy as jnp
from jax.experimental import pallas as pl
from jax.experimental.pallas import tpu as pltpu

BERT_HIDDEN = 32
HIDDEN = 16
IN_DIM = BERT_HIDDEN + HIDDEN
NHEAD = 4
DHEAD = IN_DIM // NHEAD
DIM_FF = HIDDEN
LN_EPS = 1e-5


def _node_kernel(x_ref, len_ref, pool_ref, w_ref, b_ref, out_ref):
    TN, S1, _ = x_ref.shape
    S = S1 - 1
    lens = len_ref[...]
    pos = jax.lax.broadcasted_iota(jnp.int32, (TN, S1), 1)
    valid = ((pos >= 1) & (pos <= lens)).astype(jnp.float32)[:, :, None]
    xm_sum = jnp.sum(x_ref[...] * valid, axis=1)
    node = jnp.dot(xm_sum, w_ref[...],
                   preferred_element_type=jnp.float32) + jnp.float32(S) * b_ref[...]
    out_ref[...] = jnp.concatenate([node, pool_ref[...]], axis=-1)


def _node_features(bert_last_hidden, aa_len, pooler, w, b, *, tile_n=128):
    n, S1, Hb = bert_last_hidden.shape
    n_pad = ((n + tile_n - 1) // tile_n) * tile_n
    lens = aa_len.reshape(-1, 1).astype(jnp.int32)
    if n_pad != n:
        p = n_pad - n
        bert_last_hidden = jnp.pad(bert_last_hidden, ((0, p), (0, 0), (0, 0)))
        lens = jnp.pad(lens, ((0, p), (0, 0)))
        pooler = jnp.pad(pooler, ((0, p), (0, 0)))
    out = pl.pallas_call(
        _node_kernel,
        out_shape=jax.ShapeDtypeStruct((n_pad, HIDDEN + BERT_HIDDEN), jnp.float32),
        grid_spec=pltpu.PrefetchScalarGridSpec(
            num_scalar_prefetch=0,
            grid=(n_pad // tile_n,),
            in_specs=[
                pl.BlockSpec((tile_n, S1, Hb), lambda i: (i, 0, 0)),
                pl.BlockSpec((tile_n, 1), lambda i: (i, 0)),
                pl.BlockSpec((tile_n, Hb), lambda i: (i, 0)),
                pl.BlockSpec((Hb, HIDDEN), lambda i: (0, 0)),
                pl.BlockSpec((1, HIDDEN), lambda i: (0, 0)),
            ],
            out_specs=pl.BlockSpec((tile_n, HIDDEN + BERT_HIDDEN), lambda i: (i, 0)),
        ),
        compiler_params=pltpu.CompilerParams(dimension_semantics=("parallel",)),
    )(bert_last_hidden, lens, pooler, w, b)
    return out[:n]


def _encoder_kernel(len_ref, x_ref, g_ref, w_ref, vec_ref, out_ref):
    nb, L, D = x_ref.shape

    w_qkv = w_ref[0:D, :]
    w_o = w_ref[D:2 * D, 0:D]
    w_ff1 = w_ref[2 * D:3 * D, 0:DIM_FF]
    w_ff2 = w_ref[3 * D:3 * D + DIM_FF, 0:D]
    b_qkv = vec_ref[0:1, :]
    b_o = vec_ref[1:2, 0:D]
    b_ff1 = vec_ref[2:3, 0:DIM_FF]
    b_ff2 = vec_ref[3:4, 0:D]
    ln1_g = vec_ref[4:5, 0:D]
    ln1_b = vec_ref[5:6, 0:D]
    ln2_g = vec_ref[6:7, 0:D]
    ln2_b = vec_ref[7:8, 0:D]

    xb = x_ref[...].reshape(nb * L, D)
    qkvb = jnp.dot(xb, w_qkv, preferred_element_type=jnp.float32) + b_qkv

    rh = jax.lax.broadcasted_iota(jnp.int32, (NHEAD * L, D), 0) // L
    ch = jax.lax.broadcasted_iota(jnp.int32, (NHEAD * L, D), 1) // DHEAD
    hsel = rh == ch
    r4 = jax.lax.broadcasted_iota(jnp.int32, (NHEAD * L, NHEAD), 0) // L
    c4 = jax.lax.broadcasted_iota(jnp.int32, (NHEAD * L, NHEAD), 1)
    den_ind = (r4 == c4).astype(jnp.float32)
    col = jax.lax.broadcasted_iota(jnp.int32, (L, L), 1)

    cts = []
    x2s = []
    for i in range(nb):
        ctf, ctb = _attend_one(len_ref, g_ref, qkvb[i * L:(i + 1) * L],
                               pl.program_id(0) * nb + i, i, L, D,
                               hsel, den_ind, col)
        cts.append(ctf)
        cts.append(ctb)
        xi = xb[i * L:(i + 1) * L]
        x2s.append(xi)
        x2s.append(xi)
    ct = jnp.concatenate(cts, axis=0)
    x2 = jnp.concatenate(x2s, axis=0)

    expand = (jax.lax.broadcasted_iota(jnp.int32, (NHEAD, D), 1) // DHEAD ==
              jax.lax.broadcasted_iota(jnp.int32, (NHEAD, D), 0)
              ).astype(jnp.float32)
    rec48 = jnp.dot(1.0 / ct[:, D:D + NHEAD], expand,
                    preferred_element_type=jnp.float32)
    ctx2 = ct[:, 0:D] * rec48

    attn2 = jnp.dot(ctx2, w_o, preferred_element_type=jnp.float32) + b_o

    ones_dd = jnp.full((D, D), 1.0 / D, jnp.float32)

    def layer_norm(y, gg, bb):
        mu = jnp.dot(y, ones_dd, preferred_element_type=jnp.float32)
        m2 = jnp.dot(y * y, ones_dd, preferred_element_type=jnp.float32)
        var = m2 - mu * mu
        return (y - mu) * jax.lax.rsqrt(var + LN_EPS) * gg + bb

    x1 = layer_norm(x2 + attn2, ln1_g, ln1_b)
    hidden = jax.nn.relu(
        jnp.dot(x1, w_ff1, preferred_element_type=jnp.float32) + b_ff1)
    ff = jnp.dot(hidden, w_ff2, preferred_element_type=jnp.float32) + b_ff2
    y2 = layer_norm(x1 + ff, ln2_g, ln2_b)

    for i in range(nb):
        out_ref[i] = y2[2 * i * L:(2 * i + 1) * L] + y2[(2 * i + 1) * L:(2 * i + 2) * L]


def _attend_one(len_ref, g_ref, qkv, b, i, L, D, hsel, den_ind, col):
    g = g_ref[i]
    q = qkv[:, 0:D]
    k = qkv[:, D:2 * D]
    v = qkv[:, 2 * D:3 * D]

    kt = jnp.concatenate([k, k, k, k], axis=0)
    kst = jnp.where(hsel, kt, 0.0)
    rawl = jax.lax.dot_general(q, kst, (((1,), (1,)), ((), ())),
                               preferred_element_type=jnp.float32)

    length = len_ref[b]
    cv = (col < length).astype(jnp.float32)
    m_f = jnp.where(g != 0.0, cv, 0.0).astype(jnp.bfloat16)
    m_b = jnp.where(g.T != 0.0, cv, 0.0).astype(jnp.bfloat16)

    efs = []
    ebs = []
    mxs = []
    big = jnp.float32(1e9)
    for h in range(NHEAD):
        raw_h = rawl[:, h * L:(h + 1) * L]
        mx = jnp.max(raw_h, axis=1, keepdims=True)
        e = jnp.exp(raw_h - mx).astype(jnp.bfloat16)
        efs.append(e * m_f)
        ebs.append(e * m_b)
        mxs.append(mx)
    ef = jnp.concatenate(efs, axis=1)
    eb = jnp.concatenate(ebs, axis=1)

    vt = jnp.concatenate([v, v, v, v], axis=0)
    vst = jnp.where(hsel, vt, 0.0)
    vd = jnp.concatenate([vst, den_ind], axis=1).astype(jnp.bfloat16)

    ctf = jnp.dot(ef, vd, preferred_element_type=jnp.float32)
    ctb = jnp.dot(eb, vd, preferred_element_type=jnp.float32)

    emp_f = ctf[:, D:D + 1] == 0.0
    emp_b = ctb[:, D:D + 1] == 0.0

    def _repair(cts):
        ctf, ctb = cts
        e2s = []
        for h in range(NHEAD):
            raw_h = rawl[:, h * L:(h + 1) * L]
            e2 = (raw_h - big) >= (mxs[h] - big)
            e2s.append(e2.astype(jnp.bfloat16))
        ct2 = jnp.dot(jnp.concatenate(e2s, axis=1), vd,
                      preferred_element_type=jnp.float32)
        return jnp.where(emp_f, ct2, ctf), jnp.where(emp_b, ct2, ctb)

    return jax.lax.cond(jnp.any(emp_f | emp_b), _repair,
                        lambda cts: cts, (ctf, ctb))


def _graph_encoder(graph_in, aa_graph, aa_graph_length, w_slab, vec_slab,
                   *, graphs_per_block=32):
    B, L, D = graph_in.shape
    gb = int(np.gcd(graphs_per_block, B))
    s = np.float32(1.0 / np.sqrt(DHEAD))
    wcol = jax.lax.broadcasted_iota(jnp.int32, w_slab.shape, 1)
    wrow = jax.lax.broadcasted_iota(jnp.int32, w_slab.shape, 0)
    w_slab = jnp.where((wrow < D) & (wcol < D), w_slab * s, w_slab)
    vcol = jax.lax.broadcasted_iota(jnp.int32, vec_slab.shape, 1)
    vrow = jax.lax.broadcasted_iota(jnp.int32, vec_slab.shape, 0)
    vec_slab = jnp.where((vrow == 0) & (vcol < D), vec_slab * s, vec_slab)
    return pl.pallas_call(
        _encoder_kernel,
        out_shape=jax.ShapeDtypeStruct((B, L, D), jnp.float32),
        grid_spec=pltpu.PrefetchScalarGridSpec(
            num_scalar_prefetch=1,
            grid=(B // gb,),
            in_specs=[
                pl.BlockSpec((gb, L, D), lambda b, ln: (b, 0, 0)),
                pl.BlockSpec((gb, L, L), lambda b, ln: (b, 0, 0)),
                pl.BlockSpec(w_slab.shape, lambda b, ln: (0, 0)),
                pl.BlockSpec(vec_slab.shape, lambda b, ln: (0, 0)),
            ],
            out_specs=pl.BlockSpec((gb, L, D), lambda b, ln: (b, 0, 0)),
        ),
        compiler_params=pltpu.CompilerParams(dimension_semantics=("parallel",)),
    )(aa_graph_length.astype(jnp.int32), graph_in,
      aa_graph.astype(jnp.float32), w_slab, vec_slab)


def kernel(dense_w, dense_b, w_slab, vec_slab, as_features, bert_last_hidden,
           bert_pooler, aa_len, map_AA, map_AA_idx, map_AS, map_AS_idx,
           aa_graph_length, aa_graph):
    B, L, _ = aa_graph.shape

    rela_v_inner = _node_features(bert_last_hidden, aa_len, bert_pooler,
                                  dense_w, dense_b)

    graph_in = jnp.zeros((B, L, IN_DIM), jnp.float32)
    graph_in = graph_in.at[map_AA, map_AA_idx].set(rela_v_inner)
    graph_in = graph_in.at[map_AS, map_AS_idx].set(as_features)

    mutual = _graph_encoder(graph_in, aa_graph, aa_graph_length,
                            w_slab, vec_slab)

    AA_features = mutual[map_AS, map_AS_idx]
    need = (aa_graph_length[map_AS] > 1).astype(jnp.float32)[:, None]
    blended = AA_features * need + as_features * (1.0 - need)
    return blended + as_features


# speedup vs baseline: 1.1873x; 1.1873x over previous
import numpy as np
import jax
import jax.numpy as jnp
from jax.experimental import pallas as pl
from jax.experimental.pallas import tpu as pltpu

BERT_HIDDEN = 32
HIDDEN = 16
IN_DIM = BERT_HIDDEN + HIDDEN
NHEAD = 4
DHEAD = IN_DIM // NHEAD
DIM_FF = HIDDEN
LN_EPS = 1e-5


def _node_kernel(x_ref, len_ref, pool_ref, w_ref, b_ref, out_ref):
    TN, S1, _ = x_ref.shape
    S = S1 - 1
    lens = len_ref[...]
    pos = jax.lax.broadcasted_iota(jnp.int32, (TN, S1), 1)
    valid = ((pos >= 1) & (pos <= lens)).astype(jnp.float32)[:, :, None]
    xm_sum = jnp.sum(x_ref[...] * valid, axis=1)
    node = jnp.dot(xm_sum, w_ref[...],
                   preferred_element_type=jnp.float32) + jnp.float32(S) * b_ref[...]
    out_ref[...] = jnp.concatenate([node, pool_ref[...]], axis=-1)


def _node_features(bert_last_hidden, aa_len, pooler, w, b, *, tile_n=128):
    n, S1, Hb = bert_last_hidden.shape
    n_pad = ((n + tile_n - 1) // tile_n) * tile_n
    lens = aa_len.reshape(-1, 1).astype(jnp.int32)
    if n_pad != n:
        p = n_pad - n
        bert_last_hidden = jnp.pad(bert_last_hidden, ((0, p), (0, 0), (0, 0)))
        lens = jnp.pad(lens, ((0, p), (0, 0)))
        pooler = jnp.pad(pooler, ((0, p), (0, 0)))
    out = pl.pallas_call(
        _node_kernel,
        out_shape=jax.ShapeDtypeStruct((n_pad, HIDDEN + BERT_HIDDEN), jnp.float32),
        grid_spec=pltpu.PrefetchScalarGridSpec(
            num_scalar_prefetch=0,
            grid=(n_pad // tile_n,),
            in_specs=[
                pl.BlockSpec((tile_n, S1, Hb), lambda i: (i, 0, 0)),
                pl.BlockSpec((tile_n, 1), lambda i: (i, 0)),
                pl.BlockSpec((tile_n, Hb), lambda i: (i, 0)),
                pl.BlockSpec((Hb, HIDDEN), lambda i: (0, 0)),
                pl.BlockSpec((1, HIDDEN), lambda i: (0, 0)),
            ],
            out_specs=pl.BlockSpec((tile_n, HIDDEN + BERT_HIDDEN), lambda i: (i, 0)),
        ),
        compiler_params=pltpu.CompilerParams(dimension_semantics=("parallel",)),
    )(bert_last_hidden, lens, pooler, w, b)
    return out[:n]


def _encoder_kernel(len_ref, x_ref, g_ref, w_ref, vec_ref, out_ref):
    nb, L, D = x_ref.shape

    w_qkv = w_ref[0:D, :]
    w_o = w_ref[D:2 * D, 0:D]
    w_ff1 = w_ref[2 * D:3 * D, 0:DIM_FF]
    w_ff2 = w_ref[3 * D:3 * D + DIM_FF, 0:D]
    b_qkv = vec_ref[0:1, :]
    b_o = vec_ref[1:2, 0:D]
    b_ff1 = vec_ref[2:3, 0:DIM_FF]
    b_ff2 = vec_ref[3:4, 0:D]
    ln1_g = vec_ref[4:5, 0:D]
    ln1_b = vec_ref[5:6, 0:D]
    ln2_g = vec_ref[6:7, 0:D]
    ln2_b = vec_ref[7:8, 0:D]

    xb = x_ref[...].reshape(nb * L, D)
    qkvb = jnp.dot(xb, w_qkv, preferred_element_type=jnp.float32) + b_qkv

    rh = jax.lax.broadcasted_iota(jnp.int32, (NHEAD * L, D), 0) // L
    ch = jax.lax.broadcasted_iota(jnp.int32, (NHEAD * L, D), 1) // DHEAD
    hsel = rh == ch
    r4 = jax.lax.broadcasted_iota(jnp.int32, (NHEAD * L, NHEAD), 0) // L
    c4 = jax.lax.broadcasted_iota(jnp.int32, (NHEAD * L, NHEAD), 1)
    den_ind = (r4 == c4).astype(jnp.float32)
    col = jax.lax.broadcasted_iota(jnp.int32, (L, L), 1)

    cts = []
    x2s = []
    for i in range(nb):
        ctf, ctb = _attend_one(len_ref, g_ref, qkvb[i * L:(i + 1) * L],
                               pl.program_id(0) * nb + i, i, L, D,
                               hsel, den_ind, col)
        cts.append(ctf)
        cts.append(ctb)
        xi = xb[i * L:(i + 1) * L]
        x2s.append(xi)
        x2s.append(xi)
    ct = jnp.concatenate(cts, axis=0)
    x2 = jnp.concatenate(x2s, axis=0)

    expand = (jax.lax.broadcasted_iota(jnp.int32, (NHEAD, D), 1) // DHEAD ==
              jax.lax.broadcasted_iota(jnp.int32, (NHEAD, D), 0)
              ).astype(jnp.float32)
    rec48 = jnp.dot(1.0 / ct[:, D:D + NHEAD], expand,
                    preferred_element_type=jnp.float32)
    ctx2 = ct[:, 0:D] * rec48

    attn2 = jnp.dot(ctx2, w_o, preferred_element_type=jnp.float32) + b_o

    ones_dd = jnp.full((D, D), 1.0 / D, jnp.float32)

    def layer_norm(y, gg, bb):
        mu = jnp.dot(y, ones_dd, preferred_element_type=jnp.float32)
        m2 = jnp.dot(y * y, ones_dd, preferred_element_type=jnp.float32)
        var = m2 - mu * mu
        return (y - mu) * jax.lax.rsqrt(var + LN_EPS) * gg + bb

    x1 = layer_norm(x2 + attn2, ln1_g, ln1_b)
    hidden = jax.nn.relu(
        jnp.dot(x1, w_ff1, preferred_element_type=jnp.float32) + b_ff1)
    ff = jnp.dot(hidden, w_ff2, preferred_element_type=jnp.float32) + b_ff2
    y2 = layer_norm(x1 + ff, ln2_g, ln2_b)

    for i in range(nb):
        out_ref[i] = y2[2 * i * L:(2 * i + 1) * L] + y2[(2 * i + 1) * L:(2 * i + 2) * L]


def _attend_one(len_ref, g_ref, qkv, b, i, L, D, hsel, den_ind, col):
    g = g_ref[i]
    q = qkv[:, 0:D]
    k = qkv[:, D:2 * D]
    v = qkv[:, 2 * D:3 * D]

    kt = jnp.concatenate([k, k, k, k], axis=0)
    kst = jnp.where(hsel, kt, 0.0)
    rawl = jax.lax.dot_general(q, kst, (((1,), (1,)), ((), ())),
                               preferred_element_type=jnp.float32)

    length = len_ref[b]
    cv = (col < length).astype(jnp.float32)
    m_f = jnp.where(g != 0.0, cv, 0.0).astype(jnp.bfloat16)
    m_b = jnp.where(g.T != 0.0, cv, 0.0).astype(jnp.bfloat16)

    efs = []
    ebs = []
    mxs = []
    big = jnp.float32(1e9)
    for h in range(NHEAD):
        raw_h = rawl[:, h * L:(h + 1) * L]
        mx = jnp.max(raw_h, axis=1, keepdims=True)
        e = jnp.exp(raw_h - mx).astype(jnp.bfloat16)
        efs.append(e * m_f)
        ebs.append(e * m_b)
        mxs.append(mx)
    ef = jnp.concatenate(efs, axis=1)
    eb = jnp.concatenate(ebs, axis=1)

    vt = jnp.concatenate([v, v, v, v], axis=0)
    vst = jnp.where(hsel, vt, 0.0)
    vd = jnp.concatenate([vst, den_ind], axis=1).astype(jnp.bfloat16)

    ctf = jnp.dot(ef, vd, preferred_element_type=jnp.float32)
    ctb = jnp.dot(eb, vd, preferred_element_type=jnp.float32)

    emp_f = ctf[:, D:D + 1] == 0.0
    emp_b = ctb[:, D:D + 1] == 0.0

    def _repair(cts):
        ctf, ctb = cts
        e2s = []
        for h in range(NHEAD):
            raw_h = rawl[:, h * L:(h + 1) * L]
            e2 = (raw_h - big) >= (mxs[h] - big)
            e2s.append(e2.astype(jnp.bfloat16))
        ct2 = jnp.dot(jnp.concatenate(e2s, axis=1), vd,
                      preferred_element_type=jnp.float32)
        return jnp.where(emp_f, ct2, ctf), jnp.where(emp_b, ct2, ctb)

    return jax.lax.cond(jnp.any(emp_f | emp_b), _repair,
                        lambda cts: cts, (ctf, ctb))


def _graph_encoder(graph_in, aa_graph, aa_graph_length, w_slab, vec_slab,
                   *, graphs_per_block=16):
    B, L, D = graph_in.shape
    gb = int(np.gcd(graphs_per_block, B))
    s = np.float32(1.0 / np.sqrt(DHEAD))
    wcol = jax.lax.broadcasted_iota(jnp.int32, w_slab.shape, 1)
    wrow = jax.lax.broadcasted_iota(jnp.int32, w_slab.shape, 0)
    w_slab = jnp.where((wrow < D) & (wcol < D), w_slab * s, w_slab)
    vcol = jax.lax.broadcasted_iota(jnp.int32, vec_slab.shape, 1)
    vrow = jax.lax.broadcasted_iota(jnp.int32, vec_slab.shape, 0)
    vec_slab = jnp.where((vrow == 0) & (vcol < D), vec_slab * s, vec_slab)
    return pl.pallas_call(
        _encoder_kernel,
        out_shape=jax.ShapeDtypeStruct((B, L, D), jnp.float32),
        grid_spec=pltpu.PrefetchScalarGridSpec(
            num_scalar_prefetch=1,
            grid=(B // gb,),
            in_specs=[
                pl.BlockSpec((gb, L, D), lambda b, ln: (b, 0, 0)),
                pl.BlockSpec((gb, L, L), lambda b, ln: (b, 0, 0)),
                pl.BlockSpec(w_slab.shape, lambda b, ln: (0, 0)),
                pl.BlockSpec(vec_slab.shape, lambda b, ln: (0, 0)),
            ],
            out_specs=pl.BlockSpec((gb, L, D), lambda b, ln: (b, 0, 0)),
        ),
        compiler_params=pltpu.CompilerParams(dimension_semantics=("parallel",)),
    )(aa_graph_length.astype(jnp.int32), graph_in,
      aa_graph.astype(jnp.float32), w_slab, vec_slab)


def kernel(dense_w, dense_b, w_slab, vec_slab, as_features, bert_last_hidden,
           bert_pooler, aa_len, map_AA, map_AA_idx, map_AS, map_AS_idx,
           aa_graph_length, aa_graph):
    B, L, _ = aa_graph.shape

    rela_v_inner = _node_features(bert_last_hidden, aa_len, bert_pooler,
                                  dense_w, dense_b)

    graph_in = jnp.zeros((B, L, IN_DIM), jnp.float32)
    graph_in = graph_in.at[map_AA, map_AA_idx].set(rela_v_inner)
    graph_in = graph_in.at[map_AS, map_AS_idx].set(as_features)

    mutual = _graph_encoder(graph_in, aa_graph, aa_graph_length,
                            w_slab, vec_slab)

    AA_features = mutual[map_AS, map_AS_idx]
    need = (aa_graph_length[map_AS] > 1).astype(jnp.float32)[:, None]
    blended = AA_features * need + as_features * (1.0 - need)
    return blended + as_features
